# baseline (speedup 1.0000x reference)
import functools

import jax
import jax.numpy as jnp
from jax import lax
from jax.experimental import pallas as pl
from jax.experimental.pallas import tpu as pltpu
from jax.experimental.pallas import tpu_sc as plsc

_V = 1000
_VP = 1024
_D = 64
_BATCH = 1024
_SEQ = 50
_B = _BATCH * _SEQ
_NC = 2
_NS = 16
_NW = _NC * _NS
_TPW = _B // _NW
_CH = 64
_NCHUNK = _TPW // _CH


def _fuse_table_kernel(e_ref, w_ref, b_ref, m_ref):
    m_ref[...] = lax.dot_general(
        e_ref[...], w_ref[...],
        (((1,), (1,)), ((), ())),
        preferred_element_type=jnp.float32,
    ) + b_ref[...]


def _fuse_table(embed_pad, projw_pad, projb_pad):
    return pl.pallas_call(
        _fuse_table_kernel,
        out_shape=jax.ShapeDtypeStruct((_VP, _VP), jnp.float32),
    )(embed_pad, projw_pad, projb_pad)


_mesh = plsc.VectorSubcoreMesh(core_axis_name="c", subcore_axis_name="s")


@functools.partial(
    pl.kernel,
    mesh=_mesh,
    out_type=jax.ShapeDtypeStruct((_B, _VP), jnp.float32),
    scratch_types=[
        pltpu.VMEM((_TPW,), jnp.int32),
        pltpu.VMEM((_CH, _VP), jnp.float32),
        pltpu.SemaphoreType.DMA,
    ],
)
def _gather_rows(table_hbm, ids_hbm, out_hbm, idx_v, rows_v, sem):
    wid = lax.axis_index("s") * _NC + lax.axis_index("c")
    base = wid * _TPW
    pltpu.sync_copy(ids_hbm.at[pl.ds(base, _TPW)], idx_v)

    def body(c, carry):
        off = c * _CH
        pltpu.async_copy(
            table_hbm.at[idx_v.at[pl.ds(off, _CH)]], rows_v, sem
        ).wait()
        pltpu.sync_copy(rows_v, out_hbm.at[pl.ds(base + off, _CH)])
        return carry

    lax.fori_loop(0, _NCHUNK, body, 0)


def kernel(input_ids, embed_table, proj_w, proj_b):
    embed_pad = jnp.pad(embed_table, ((0, _VP - _V), (0, 0)))
    projw_pad = jnp.pad(proj_w, ((0, _VP - _V), (0, 0)))
    projb_pad = jnp.pad(proj_b, (0, _VP - _V)).reshape(1, _VP)
    m = _fuse_table(embed_pad, projw_pad, projb_pad)
    ids = input_ids.reshape(_B).astype(jnp.int32)
    out = _gather_rows(m, ids)
    return out[:, :_V].reshape(_BATCH, _SEQ, _V)


# speedup vs baseline: 1.0099x; 1.0099x over previous
import functools

import jax
import jax.numpy as jnp
from jax import lax
from jax.experimental import pallas as pl
from jax.experimental.pallas import tpu as pltpu
from jax.experimental.pallas import tpu_sc as plsc

_V = 1000
_VP = 1024
_D = 64
_BATCH = 1024
_SEQ = 50
_B = _BATCH * _SEQ
_NC = 2
_NS = 16
_NW = _NC * _NS
_TPW = _B // _NW
_CH = 40
_NCHUNK = _TPW // _CH


def _fuse_table_kernel(e_ref, w_ref, b_ref, m_ref):
    m_ref[...] = lax.dot_general(
        e_ref[...], w_ref[...],
        (((1,), (1,)), ((), ())),
        preferred_element_type=jnp.float32,
    ) + b_ref[...]


def _fuse_table(embed_pad, projw_pad, projb_pad):
    return pl.pallas_call(
        _fuse_table_kernel,
        out_shape=jax.ShapeDtypeStruct((_VP, _VP), jnp.float32),
    )(embed_pad, projw_pad, projb_pad)


_mesh = plsc.VectorSubcoreMesh(core_axis_name="c", subcore_axis_name="s")


@functools.partial(
    pl.kernel,
    mesh=_mesh,
    out_type=jax.ShapeDtypeStruct((_B, _VP), jnp.float32),
    scratch_types=[
        pltpu.VMEM((_TPW,), jnp.int32),
        pltpu.VMEM((2, _CH, _VP), jnp.float32),
        pltpu.SemaphoreType.DMA,
        pltpu.SemaphoreType.DMA,
        pltpu.SemaphoreType.DMA,
        pltpu.SemaphoreType.DMA,
    ],
)
def _gather_rows(table_hbm, ids_hbm, out_hbm, idx_v, rows_v, gs0, gs1,
                 ss0, ss1):
    wid = lax.axis_index("s") * _NC + lax.axis_index("c")
    base = wid * _TPW
    pltpu.sync_copy(ids_hbm.at[pl.ds(base, _TPW)], idx_v)

    def gather(c, p):
        gsem = gs0 if p == 0 else gs1
        return pltpu.make_async_copy(
            table_hbm.at[idx_v.at[pl.ds(c * _CH, _CH)]],
            rows_v.at[p], gsem)

    def scatter(c, p):
        ssem = ss0 if p == 0 else ss1
        return pltpu.make_async_copy(
            rows_v.at[p], out_hbm.at[pl.ds(base + c * _CH, _CH)], ssem)

    gather(0, 0).start()

    def body(i, carry):
        c0 = i * 2
        gather(c0, 0).wait()
        scatter(c0, 0).start()

        @pl.when(i >= 1)
        def _():
            scatter(c0 - 1, 1).wait()

        gather(c0 + 1, 1).start()

        c1 = c0 + 1
        gather(c1, 1).wait()
        scatter(c1, 1).start()

        @pl.when(c1 + 1 < _NCHUNK)
        def _():
            scatter(c1 - 1, 0).wait()
            gather(c1 + 1, 0).start()

        return carry

    lax.fori_loop(0, _NCHUNK // 2, body, 0)
    scatter(_NCHUNK - 2, 0).wait()
    scatter(_NCHUNK - 1, 1).wait()


def kernel(input_ids, embed_table, proj_w, proj_b):
    embed_pad = jnp.pad(embed_table, ((0, _VP - _V), (0, 0)))
    projw_pad = jnp.pad(proj_w, ((0, _VP - _V), (0, 0)))
    projb_pad = jnp.pad(proj_b, (0, _VP - _V)).reshape(1, _VP)
    m = _fuse_table(embed_pad, projw_pad, projb_pad)
    ids = input_ids.reshape(_B).astype(jnp.int32)
    out = _gather_rows(m, ids)
    return out[:, :_V].reshape(_BATCH, _SEQ, _V)


# speedup vs baseline: 1.4543x; 1.4401x over previous
import functools

import jax
import jax.numpy as jnp
from jax import lax
from jax.experimental import pallas as pl
from jax.experimental.pallas import tpu as pltpu
from jax.experimental.pallas import tpu_sc as plsc

_V = 1000
_VP = 1024
_D = 64
_BATCH = 1024
_SEQ = 50
_B = _BATCH * _SEQ
_NC = 2
_NS = 16
_NW = _NC * _NS
_BPW = _BATCH // _NW
_BBLK = 512


def _fuse_table_kernel(e_ref, w_ref, b_ref, m_ref):
    m_ref[...] = lax.dot_general(
        e_ref[...], w_ref[...],
        (((1,), (1,)), ((), ())),
        preferred_element_type=jnp.float32,
    ) + b_ref[...]


def _fuse_table(embed_pad, projw_pad, projb_pad):
    return pl.pallas_call(
        _fuse_table_kernel,
        out_shape=jax.ShapeDtypeStruct((_VP, _VP), jnp.float32),
    )(embed_pad, projw_pad, projb_pad)


_mesh = plsc.VectorSubcoreMesh(core_axis_name="c", subcore_axis_name="s")


@functools.partial(
    pl.kernel,
    mesh=_mesh,
    out_type=jax.ShapeDtypeStruct((_B, _VP), jnp.float32),
    scratch_types=[
        pltpu.VMEM((_SEQ * _BPW,), jnp.int32),
        pltpu.VMEM((2, _BPW, _VP), jnp.float32),
        pltpu.SemaphoreType.DMA,
        pltpu.SemaphoreType.DMA,
        pltpu.SemaphoreType.DMA,
        pltpu.SemaphoreType.DMA,
    ],
)
def _gather_rows(table_hbm, ids_hbm, out_hbm, idx_v, rows_v, gs0, gs1,
                 ss0, ss1):
    wid = lax.axis_index("s") * _NC + lax.axis_index("c")
    pltpu.sync_copy(ids_hbm.at[pl.ds(wid * _SEQ * _BPW, _SEQ * _BPW)],
                    idx_v)

    def gather(t, p):
        gsem = gs0 if p == 0 else gs1
        return pltpu.make_async_copy(
            table_hbm.at[idx_v.at[pl.ds(t * _BPW, _BPW)]],
            rows_v.at[p], gsem)

    def scatter(t, p):
        ssem = ss0 if p == 0 else ss1
        return pltpu.make_async_copy(
            rows_v.at[p],
            out_hbm.at[pl.ds(t * _BATCH + wid * _BPW, _BPW)], ssem)

    gather(0, 0).start()

    def body(i, carry):
        t0 = i * 2
        gather(t0, 0).wait()
        scatter(t0, 0).start()

        @pl.when(i >= 1)
        def _():
            scatter(t0 - 1, 1).wait()

        gather(t0 + 1, 1).start()

        t1 = t0 + 1
        gather(t1, 1).wait()
        scatter(t1, 1).start()

        @pl.when(t1 + 1 < _SEQ)
        def _():
            scatter(t1 - 1, 0).wait()
            gather(t1 + 1, 0).start()

        return carry

    lax.fori_loop(0, _SEQ // 2, body, 0)
    scatter(_SEQ - 2, 0).wait()
    scatter(_SEQ - 1, 1).wait()


def _transpose_kernel(g_ref, o_ref):
    o_ref[0] = jnp.swapaxes(g_ref[0], 0, 1)[:_V, :]


def _transpose_slabs(g3):
    return pl.pallas_call(
        _transpose_kernel,
        grid=(_SEQ, _BATCH // _BBLK),
        in_specs=[pl.BlockSpec((1, _BBLK, _VP), lambda t, j: (t, j, 0))],
        out_specs=pl.BlockSpec((1, _V, _BBLK), lambda t, j: (t, 0, j)),
        out_shape=jax.ShapeDtypeStruct((_SEQ, _V, _BATCH), jnp.float32),
    )(g3)


def kernel(input_ids, embed_table, proj_w, proj_b):
    embed_pad = jnp.pad(embed_table, ((0, _VP - _V), (0, 0)))
    projw_pad = jnp.pad(proj_w, ((0, _VP - _V), (0, 0)))
    projb_pad = jnp.pad(proj_b, (0, _VP - _V)).reshape(1, _VP)
    m = _fuse_table(embed_pad, projw_pad, projb_pad)
    ids = (input_ids.astype(jnp.int32)
           .reshape(_NW, _BPW, _SEQ)
           .transpose(0, 2, 1)
           .reshape(_B))
    g = _gather_rows(m, ids)
    out_t = _transpose_slabs(g.reshape(_SEQ, _BATCH, _VP))
    return out_t.transpose(2, 0, 1)


# speedup vs baseline: 1.5213x; 1.0460x over previous
import functools

import jax
import jax.numpy as jnp
from jax import lax
from jax.experimental import pallas as pl
from jax.experimental.pallas import tpu as pltpu
from jax.experimental.pallas import tpu_sc as plsc

_V = 1000
_VP = 1024
_D = 64
_BATCH = 1024
_SEQ = 50
_B = _BATCH * _SEQ
_NC = 2
_NS = 16
_NW = _NC * _NS
_BPW = _BATCH // _NW
_BBLK = 512
_NCHK = 5
_TCH = _SEQ // _NCHK


def _fuse_table_kernel(e_ref, w_ref, b_ref, m_ref):
    m_ref[...] = lax.dot_general(
        e_ref[...], w_ref[...],
        (((1,), (1,)), ((), ())),
        preferred_element_type=jnp.float32,
    ) + b_ref[...]


def _fuse_table(embed_pad, projw_pad, projb_pad):
    return pl.pallas_call(
        _fuse_table_kernel,
        out_shape=jax.ShapeDtypeStruct((_VP, _VP), jnp.float32),
    )(embed_pad, projw_pad, projb_pad)


_mesh = plsc.VectorSubcoreMesh(core_axis_name="c", subcore_axis_name="s")


def _make_gather(k):
    @functools.partial(
        pl.kernel,
        mesh=_mesh,
        out_type=jax.ShapeDtypeStruct((_TCH * _BATCH, _VP), jnp.float32),
        scratch_types=[
            pltpu.VMEM((_TCH * _BPW,), jnp.int32),
            pltpu.VMEM((2, _BPW, _VP), jnp.float32),
            pltpu.SemaphoreType.DMA,
            pltpu.SemaphoreType.DMA,
            pltpu.SemaphoreType.DMA,
            pltpu.SemaphoreType.DMA,
        ],
    )
    def _gather_chunk(table_hbm, ids_hbm, out_hbm, idx_v, rows_v,
                      gs0, gs1, ss0, ss1):
        wid = lax.axis_index("s") * _NC + lax.axis_index("c")
        pltpu.sync_copy(
            ids_hbm.at[pl.ds((k * _NW + wid) * _TCH * _BPW,
                             _TCH * _BPW)], idx_v)

        def gather(t, p):
            gsem = gs0 if p == 0 else gs1
            return pltpu.make_async_copy(
                table_hbm.at[idx_v.at[pl.ds(t * _BPW, _BPW)]],
                rows_v.at[p], gsem)

        def scatter(t, p):
            ssem = ss0 if p == 0 else ss1
            return pltpu.make_async_copy(
                rows_v.at[p],
                out_hbm.at[pl.ds(t * _BATCH + wid * _BPW, _BPW)], ssem)

        gather(0, 0).start()

        def body(i, carry):
            t0 = i * 2
            gather(t0, 0).wait()
            scatter(t0, 0).start()

            @pl.when(i >= 1)
            def _():
                scatter(t0 - 1, 1).wait()

            gather(t0 + 1, 1).start()

            t1 = t0 + 1
            gather(t1, 1).wait()
            scatter(t1, 1).start()

            @pl.when(t1 + 1 < _TCH)
            def _():
                scatter(t1 - 1, 0).wait()
                gather(t1 + 1, 0).start()

            return carry

        lax.fori_loop(0, _TCH // 2, body, 0)
        scatter(_TCH - 2, 0).wait()
        scatter(_TCH - 1, 1).wait()

    return _gather_chunk


_gather_chunks = [_make_gather(k) for k in range(_NCHK)]


def _transpose_first_kernel(g_ref, o_ref):
    o_ref[0] = jnp.swapaxes(g_ref[0], 0, 1)[:_V, :]


def _transpose_next_kernel(g_ref, prev_ref, o_ref):
    del prev_ref
    o_ref[0] = jnp.swapaxes(g_ref[0], 0, 1)[:_V, :]


def _transpose_chunk(k, g3, prev):
    g_spec = pl.BlockSpec((1, _BBLK, _VP), lambda t, j: (t, j, 0))
    o_spec = pl.BlockSpec((1, _V, _BBLK),
                          lambda t, j, k=k: (k * _TCH + t, 0, j))
    out_shape = jax.ShapeDtypeStruct((_SEQ, _V, _BATCH), jnp.float32)
    grid = (_TCH, _BATCH // _BBLK)
    if prev is None:
        return pl.pallas_call(
            _transpose_first_kernel, grid=grid,
            in_specs=[g_spec], out_specs=o_spec, out_shape=out_shape,
        )(g3)
    return pl.pallas_call(
        _transpose_next_kernel, grid=grid,
        in_specs=[g_spec,
                  pl.BlockSpec(memory_space=pltpu.MemorySpace.HBM)],
        out_specs=o_spec, out_shape=out_shape,
        input_output_aliases={1: 0},
    )(g3, prev)


def kernel(input_ids, embed_table, proj_w, proj_b):
    embed_pad = jnp.pad(embed_table, ((0, _VP - _V), (0, 0)))
    projw_pad = jnp.pad(proj_w, ((0, _VP - _V), (0, 0)))
    projb_pad = jnp.pad(proj_b, (0, _VP - _V)).reshape(1, _VP)
    m = _fuse_table(embed_pad, projw_pad, projb_pad)
    ids = (input_ids.astype(jnp.int32)
           .reshape(_NW, _BPW, _NCHK, _TCH)
           .transpose(2, 0, 3, 1)
           .reshape(_B))
    gs = [g(m, ids) for g in _gather_chunks]
    out_t = None
    for k in range(_NCHK):
        g3 = gs[k].reshape(_TCH, _BATCH, _VP)
        out_t = _transpose_chunk(k, g3, out_t)
    return out_t.transpose(2, 0, 1)


# speedup vs baseline: 3.1776x; 2.0887x over previous
import functools

import jax
import jax.numpy as jnp
from jax import lax
from jax.experimental import pallas as pl
from jax.experimental.pallas import tpu as pltpu
from jax.experimental.pallas import tpu_sc as plsc

_V = 1000
_D = 64
_DP = 128
_BATCH = 1024
_SEQ = 50
_B = _BATCH * _SEQ
_NC = 2
_NS = 16
_NW = _NC * _NS
_BPW = _BATCH // _NW

_mesh = plsc.VectorSubcoreMesh(core_axis_name="c", subcore_axis_name="s")


@functools.partial(
    pl.kernel,
    mesh=_mesh,
    out_type=jax.ShapeDtypeStruct((_B, _DP), jnp.float32),
    scratch_types=[
        pltpu.VMEM((_SEQ * _BPW,), jnp.int32),
        pltpu.VMEM((2, _BPW, _DP), jnp.float32),
        pltpu.SemaphoreType.DMA,
        pltpu.SemaphoreType.DMA,
        pltpu.SemaphoreType.DMA,
        pltpu.SemaphoreType.DMA,
    ],
)
def _gather_x(table_hbm, ids_hbm, out_hbm, idx_v, rows_v, gs0, gs1,
              ss0, ss1):
    wid = lax.axis_index("s") * _NC + lax.axis_index("c")
    pltpu.sync_copy(ids_hbm.at[pl.ds(wid * _SEQ * _BPW, _SEQ * _BPW)],
                    idx_v)

    def gather(t, p):
        gsem = gs0 if p == 0 else gs1
        return pltpu.make_async_copy(
            table_hbm.at[idx_v.at[pl.ds(t * _BPW, _BPW)]],
            rows_v.at[p], gsem)

    def scatter(t, p):
        ssem = ss0 if p == 0 else ss1
        return pltpu.make_async_copy(
            rows_v.at[p],
            out_hbm.at[pl.ds(t * _BATCH + wid * _BPW, _BPW)], ssem)

    gather(0, 0).start()

    def body(i, carry):
        t0 = i * 2
        gather(t0, 0).wait()
        scatter(t0, 0).start()

        @pl.when(i >= 1)
        def _():
            scatter(t0 - 1, 1).wait()

        gather(t0 + 1, 1).start()

        t1 = t0 + 1
        gather(t1, 1).wait()
        scatter(t1, 1).start()

        @pl.when(t1 + 1 < _SEQ)
        def _():
            scatter(t1 - 1, 0).wait()
            gather(t1 + 1, 0).start()

        return carry

    lax.fori_loop(0, _SEQ // 2, body, 0)
    scatter(_SEQ - 2, 0).wait()
    scatter(_SEQ - 1, 1).wait()


def _proj_kernel(x_ref, w_ref, b_ref, o_ref):
    o_ref[0] = lax.dot_general(
        w_ref[...], x_ref[0],
        (((1,), (1,)), ((), ())),
        preferred_element_type=jnp.float32,
    ) + b_ref[...]


def _project(x3, w_pad, b2d):
    return pl.pallas_call(
        _proj_kernel,
        grid=(_SEQ,),
        in_specs=[
            pl.BlockSpec((1, _BATCH, _DP), lambda t: (t, 0, 0)),
            pl.BlockSpec((_V, _DP), lambda t: (0, 0)),
            pl.BlockSpec((_V, 1), lambda t: (0, 0)),
        ],
        out_specs=pl.BlockSpec((1, _V, _BATCH), lambda t: (t, 0, 0)),
        out_shape=jax.ShapeDtypeStruct((_SEQ, _V, _BATCH), jnp.float32),
    )(x3, w_pad, b2d)


def kernel(input_ids, embed_table, proj_w, proj_b):
    embed_pad = jnp.pad(embed_table, ((0, 0), (0, _DP - _D)))
    w_pad = jnp.pad(proj_w, ((0, 0), (0, _DP - _D)))
    b2d = proj_b.reshape(_V, 1)
    ids = (input_ids.astype(jnp.int32)
           .reshape(_NW, _BPW, _SEQ)
           .transpose(0, 2, 1)
           .reshape(_B))
    x = _gather_x(embed_pad, ids)
    out_t = _project(x.reshape(_SEQ, _BATCH, _DP), w_pad, b2d)
    return out_t.transpose(2, 0, 1)


# speedup vs baseline: 3.5879x; 1.1291x over previous
import functools

import jax
import jax.numpy as jnp
from jax import lax
from jax.experimental import pallas as pl
from jax.experimental.pallas import tpu as pltpu
from jax.experimental.pallas import tpu_sc as plsc

_V = 1000
_D = 64
_DP = 128
_BATCH = 1024
_SEQ = 50
_B = _BATCH * _SEQ
_NC = 2
_NS = 16
_NW = _NC * _NS
_TPW = _B // _NW
_CH = 80
_NCHUNK = _TPW // _CH

_mesh = plsc.VectorSubcoreMesh(core_axis_name="c", subcore_axis_name="s")


@functools.partial(
    pl.kernel,
    mesh=_mesh,
    out_type=jax.ShapeDtypeStruct((_B, _DP), jnp.float32),
    scratch_types=[
        pltpu.VMEM((_TPW,), jnp.int32),
        pltpu.VMEM((2, _CH, _DP), jnp.float32),
        pltpu.SemaphoreType.DMA,
        pltpu.SemaphoreType.DMA,
        pltpu.SemaphoreType.DMA,
        pltpu.SemaphoreType.DMA,
    ],
)
def _gather_x(table_hbm, ids_hbm, out_hbm, idx_v, rows_v, gs0, gs1,
              ss0, ss1):
    wid = lax.axis_index("s") * _NC + lax.axis_index("c")
    base = wid * _TPW
    pltpu.sync_copy(ids_hbm.at[pl.ds(base, _TPW)], idx_v)

    def gather(c, p):
        gsem = gs0 if p == 0 else gs1
        return pltpu.make_async_copy(
            table_hbm.at[idx_v.at[pl.ds(c * _CH, _CH)]],
            rows_v.at[p], gsem)

    def scatter(c, p):
        ssem = ss0 if p == 0 else ss1
        return pltpu.make_async_copy(
            rows_v.at[p], out_hbm.at[pl.ds(base + c * _CH, _CH)], ssem)

    gather(0, 0).start()

    def body(i, carry):
        c0 = i * 2
        gather(c0, 0).wait()
        scatter(c0, 0).start()

        @pl.when(i >= 1)
        def _():
            scatter(c0 - 1, 1).wait()

        gather(c0 + 1, 1).start()

        c1 = c0 + 1
        gather(c1, 1).wait()
        scatter(c1, 1).start()

        @pl.when(c1 + 1 < _NCHUNK)
        def _():
            scatter(c1 - 1, 0).wait()
            gather(c1 + 1, 0).start()

        return carry

    lax.fori_loop(0, _NCHUNK // 2, body, 0)
    scatter(_NCHUNK - 2, 0).wait()
    scatter(_NCHUNK - 1, 1).wait()


def _proj_kernel(x_ref, w_ref, b_ref, o_ref):
    o_ref[0] = lax.dot_general(
        w_ref[...], x_ref[0],
        (((1,), (1,)), ((), ())),
        preferred_element_type=jnp.float32,
    ) + b_ref[...]


def _project(x3, w_pad, b2d):
    return pl.pallas_call(
        _proj_kernel,
        grid=(_SEQ,),
        in_specs=[
            pl.BlockSpec((1, _BATCH, _DP), lambda t: (t, 0, 0)),
            pl.BlockSpec((_V, _DP), lambda t: (0, 0)),
            pl.BlockSpec((_V, 1), lambda t: (0, 0)),
        ],
        out_specs=pl.BlockSpec((1, _V, _BATCH), lambda t: (t, 0, 0)),
        out_shape=jax.ShapeDtypeStruct((_SEQ, _V, _BATCH), jnp.float32),
    )(x3, w_pad, b2d)


def kernel(input_ids, embed_table, proj_w, proj_b):
    embed_pad = jnp.pad(embed_table, ((0, 0), (0, _DP - _D)))
    w_pad = jnp.pad(proj_w, ((0, 0), (0, _DP - _D)))
    b2d = proj_b.reshape(_V, 1)
    ids = input_ids.astype(jnp.int32).T.reshape(_B)
    x = _gather_x(embed_pad, ids)
    out_t = _project(x.reshape(_SEQ, _BATCH, _DP), w_pad, b2d)
    return out_t.transpose(2, 0, 1)
